# baseline (speedup 1.0000x reference)
import functools

import jax
import jax.numpy as jnp
from jax import lax
from jax.experimental import pallas as pl
from jax.experimental.pallas import tpu as pltpu
from jax.experimental.pallas import tpu_sc as plsc

D_MODEL = 32
_ROWS, _COLS = 16384, 26
_B = _ROWS * _COLS
_L = 128
_NCHUNK = 4
_U = _NCHUNK * _L
_UPR = _ROWS // _U
_NW = 32
_NU = _COLS * _UPR // _NW


def _make_gather():
    mesh = plsc.VectorSubcoreMesh(core_axis_name="c", subcore_axis_name="s")

    @functools.partial(
        pl.kernel,
        out_type=jax.ShapeDtypeStruct((_B, D_MODEL), jnp.float32),
        mesh=mesh,
        scratch_types=[
            pltpu.VMEM((2, _U), jnp.int32),
            pltpu.VMEM((2, _U, D_MODEL), jnp.float32),
            pltpu.SemaphoreType.DMA,
            pltpu.SemaphoreType.DMA,
            pltpu.SemaphoreType.DMA,
            pltpu.SemaphoreType.DMA,
        ],
        compiler_params=pltpu.CompilerParams(use_tc_tiling_on_sc=False),
    )
    def gather(table_hbm, idx_hbm, out_hbm, idx_v, rows_v, gsem0, gsem1,
               osem0, osem1):
        wid = lax.axis_index("s") * 2 + lax.axis_index("c")
        ubase = wid * _NU
        gsems = (gsem0, gsem1)
        osems = (osem0, osem1)

        @pl.loop(0, _NU, step=2)
        def _pair(uo):
            for b in range(2):
                u = ubase + uo + b
                j = u // _UPR
                i0 = (u % _UPR) * _U

                @pl.when(uo != 0)
                def _reclaim():
                    pltpu.make_async_copy(
                        rows_v.at[b],
                        out_hbm.at[pl.ds(j * _ROWS + i0, _U)],
                        osems[b]).wait()

                pltpu.sync_copy(idx_hbm.at[j, pl.ds(i0, _U)], idx_v.at[b])
                for k in range(_NCHUNK):
                    pltpu.async_copy(
                        table_hbm.at[idx_v.at[b, pl.ds(k * _L, _L)]],
                        rows_v.at[b, pl.ds(k * _L, _L)],
                        gsems[b])
            for b in range(2):
                u = ubase + uo + b
                j = u // _UPR
                i0 = (u % _UPR) * _U
                for k in range(_NCHUNK):
                    pltpu.make_async_copy(
                        table_hbm.at[idx_v.at[b, pl.ds(k * _L, _L)]],
                        rows_v.at[b, pl.ds(k * _L, _L)],
                        gsems[b]).wait()
                pltpu.async_copy(rows_v.at[b],
                                 out_hbm.at[pl.ds(j * _ROWS + i0, _U)],
                                 osems[b])

        for b in range(2):
            u = ubase + _NU - 2 + b
            j = u // _UPR
            i0 = (u % _UPR) * _U
            pltpu.make_async_copy(
                rows_v.at[b],
                out_hbm.at[pl.ds(j * _ROWS + i0, _U)],
                osems[b]).wait()

    return gather


_gather = _make_gather()


@jax.jit
def kernel(x, embed_weight):
    idx = jnp.transpose(x).astype(jnp.int32)
    out = _gather(embed_weight, idx)
    return jnp.transpose(out.reshape(_COLS, _ROWS, D_MODEL), (1, 0, 2))


# speedup vs baseline: 1.0002x; 1.0002x over previous
import functools

import jax
import jax.numpy as jnp
from jax import lax
from jax.experimental import pallas as pl
from jax.experimental.pallas import tpu as pltpu
from jax.experimental.pallas import tpu_sc as plsc

D_MODEL = 32
_ROWS, _COLS = 16384, 26
_B = _ROWS * _COLS
_L = 128
_NCHUNK = 4
_U = _NCHUNK * _L
_UPR = _ROWS // _U
_NW = 32
_NU = _COLS * _UPR // _NW


def _make_gather():
    mesh = plsc.VectorSubcoreMesh(core_axis_name="c", subcore_axis_name="s")

    @functools.partial(
        pl.kernel,
        out_type=jax.ShapeDtypeStruct((_COLS, _ROWS, D_MODEL), jnp.float32),
        mesh=mesh,
        scratch_types=[
            pltpu.VMEM((2, _U), jnp.float32),
            pltpu.VMEM((2, _U), jnp.int32),
            pltpu.VMEM((2, _U, D_MODEL), jnp.float32),
            pltpu.SemaphoreType.DMA,
            pltpu.SemaphoreType.DMA,
            pltpu.SemaphoreType.DMA,
            pltpu.SemaphoreType.DMA,
        ],
        compiler_params=pltpu.CompilerParams(use_tc_tiling_on_sc=False,
                                             needs_layout_passes=False),
    )
    def gather(table_hbm, idx_hbm, out_hbm, idxf_v, idx_v, rows_v,
               gsem0, gsem1, osem0, osem1):
        wid = lax.axis_index("s") * 2 + lax.axis_index("c")
        ubase = wid * _NU
        gsems = (gsem0, gsem1)
        osems = (osem0, osem1)

        @pl.loop(0, _NU, step=2)
        def _pair(uo):
            for b in range(2):
                u = ubase + uo + b
                j = u // _UPR
                i0 = (u % _UPR) * _U

                @pl.when(uo != 0)
                def _reclaim():
                    pltpu.make_async_copy(
                        rows_v.at[b],
                        out_hbm.at[j, pl.ds(i0, _U)],
                        osems[b]).wait()

                pltpu.sync_copy(idx_hbm.at[j, pl.ds(i0, _U)], idxf_v.at[b])
                for k in range(_U // 16):
                    idx_v[b, pl.ds(k * 16, 16)] = plsc.bitcast(
                        idxf_v[b, pl.ds(k * 16, 16)], jnp.int32)
                for k in range(_NCHUNK):
                    pltpu.async_copy(
                        table_hbm.at[idx_v.at[b, pl.ds(k * _L, _L)]],
                        rows_v.at[b, pl.ds(k * _L, _L)],
                        gsems[b])
            for b in range(2):
                u = ubase + uo + b
                j = u // _UPR
                i0 = (u % _UPR) * _U
                for k in range(_NCHUNK):
                    pltpu.make_async_copy(
                        table_hbm.at[idx_v.at[b, pl.ds(k * _L, _L)]],
                        rows_v.at[b, pl.ds(k * _L, _L)],
                        gsems[b]).wait()
                pltpu.async_copy(rows_v.at[b],
                                 out_hbm.at[j, pl.ds(i0, _U)],
                                 osems[b])

        for b in range(2):
            u = ubase + _NU - 2 + b
            j = u // _UPR
            i0 = (u % _UPR) * _U
            pltpu.make_async_copy(
                rows_v.at[b],
                out_hbm.at[j, pl.ds(i0, _U)],
                osems[b]).wait()

    return gather


_gather = _make_gather()


@jax.jit
def kernel(x, embed_weight):
    idx = jnp.transpose(lax.bitcast_convert_type(x.astype(jnp.int32),
                                                 jnp.float32))
    out = _gather(embed_weight, idx)
    return jnp.transpose(out, (1, 0, 2))
